# initial kernel scaffold (unmeasured)
import functools

import jax
import jax.numpy as jnp
from jax import lax
from jax.experimental import pallas as pl
from jax.experimental.pallas import tpu as pltpu

M = 4096
D = 4096
CH = 512
NCH = M // CH


def _body(mine_ref, theirs_ref, gamma_ref, out_ref,
          recv_buf, send_sems, recv_sems):
    i = pl.program_id(0)
    slot = lax.rem(i, 2)

    px = 1 - lax.axis_index("x")
    my_y = lax.axis_index("y")
    my_z = lax.axis_index("z")
    partner = (px, my_y, my_z)

    @pl.when(i == 0)
    def _():
        barrier = pltpu.get_barrier_semaphore()
        pl.semaphore_signal(barrier, inc=1, device_id=partner,
                            device_id_type=pl.DeviceIdType.MESH)
        pl.semaphore_wait(barrier, 1)

    rdma = pltpu.make_async_remote_copy(
        src_ref=theirs_ref,
        dst_ref=recv_buf.at[slot],
        send_sem=send_sems.at[slot],
        recv_sem=recv_sems.at[slot],
        device_id=partner,
        device_id_type=pl.DeviceIdType.MESH,
    )
    rdma.start()
    rdma.wait()

    y = mine_ref[:, :] + recv_buf[slot].astype(jnp.float32)
    ms = jnp.mean(y * y, axis=-1, keepdims=True)
    out_ref[:, :] = y * lax.rsqrt(ms + 1e-6) * gamma_ref[:, :]


def kernel(partial, gamma):
    me = lax.axis_index("x")
    halves = partial.reshape(2, M, D)
    mine = lax.dynamic_index_in_dim(halves, me, 0, keepdims=False)
    theirs = lax.dynamic_index_in_dim(halves, 1 - me, 0, keepdims=False)
    theirs = theirs.astype(jnp.bfloat16)
    gamma2 = gamma.reshape(1, D)

    return pl.pallas_call(
        _body,
        grid=(NCH,),
        in_specs=[
            pl.BlockSpec((CH, D), lambda i: (i, 0)),
            pl.BlockSpec((CH, D), lambda i: (i, 0)),
            pl.BlockSpec((1, D), lambda i: (0, 0)),
        ],
        out_specs=pl.BlockSpec((CH, D), lambda i: (i, 0)),
        out_shape=jax.ShapeDtypeStruct((M, D), jnp.float32),
        scratch_shapes=[
            pltpu.VMEM((2, CH, D), jnp.bfloat16),
            pltpu.SemaphoreType.DMA((2,)),
            pltpu.SemaphoreType.DMA((2,)),
        ],
        compiler_params=pltpu.CompilerParams(collective_id=0),
    )(mine, theirs, gamma2)


# baseline (device time: 527257 ns/iter reference)
import functools

import jax
import jax.numpy as jnp
from jax import lax
from jax.experimental import pallas as pl
from jax.experimental.pallas import tpu as pltpu

M = 4096
D = 4096
CH = 512
NCH = M // CH


def _body(mine_ref, theirs_ref, gamma_ref, out_ref,
          recv_buf, send_sems, recv_sems):
    i = pl.program_id(0)
    slot = lax.rem(i, 2)

    px = 1 - lax.axis_index("x")
    my_y = lax.axis_index("y")
    my_z = lax.axis_index("z")
    partner = (px, my_y, my_z)

    @pl.when(i == 0)
    def _():
        barrier = pltpu.get_barrier_semaphore()
        pl.semaphore_signal(barrier, inc=1, device_id=partner,
                            device_id_type=pl.DeviceIdType.MESH)
        pl.semaphore_wait(barrier, 1)

    rdma = pltpu.make_async_remote_copy(
        src_ref=theirs_ref,
        dst_ref=recv_buf.at[slot],
        send_sem=send_sems.at[slot],
        recv_sem=recv_sems.at[slot],
        device_id=partner,
        device_id_type=pl.DeviceIdType.MESH,
    )
    rdma.start()
    rdma.wait()

    y = mine_ref[:, :] + recv_buf[slot].astype(jnp.float32)
    ms = jnp.mean(y * y, axis=-1, keepdims=True)
    out_ref[:, :] = y * lax.rsqrt(ms + 1e-6) * gamma_ref[:, :]


def kernel(partial, gamma):
    me = lax.axis_index("x")
    halves = partial.reshape(2, M, D)
    mine = lax.dynamic_index_in_dim(halves, me, 0, keepdims=False)
    theirs = lax.dynamic_index_in_dim(halves, 1 - me, 0, keepdims=False)
    theirs = theirs.astype(jnp.bfloat16)
    gamma2 = gamma.reshape(1, D)

    return pl.pallas_call(
        _body,
        grid=(NCH,),
        in_specs=[
            pl.BlockSpec((CH, D), lambda i: (i, 0)),
            pl.BlockSpec((CH, D), lambda i: (i, 0)),
            pl.BlockSpec((1, D), lambda i: (0, 0)),
        ],
        out_specs=pl.BlockSpec((CH, D), lambda i: (i, 0)),
        out_shape=jax.ShapeDtypeStruct((M, D), jnp.float32),
        scratch_shapes=[
            pltpu.VMEM((2, CH, D), jnp.bfloat16),
            pltpu.SemaphoreType.DMA((2,)),
            pltpu.SemaphoreType.DMA((2,)),
        ],
        compiler_params=pltpu.CompilerParams(
            collective_id=0,
            vmem_limit_bytes=100 * 1024 * 1024,
        ),
    )(mine, theirs, gamma2)


# device time: 252608 ns/iter; 2.0873x vs baseline; 2.0873x over previous
import jax
import jax.numpy as jnp
from jax import lax
from jax.experimental import pallas as pl
from jax.experimental.pallas import tpu as pltpu

M = 4096
D = 4096
HALF = M // 2
CH = 256
NC = HALF // CH
NSTAGE = 4


def _body(partial_ref, theirs_ref, gamma_ref, out_ref,
          mine_buf, recv_x, out_stage,
          mine_sems, send_x_sems, recv_x_sems,
          out_copy_sems, send_y_sems, recv_y_sems):
    my_x = lax.axis_index("x")
    my_y = lax.axis_index("y")
    my_z = lax.axis_index("z")
    xp = (1 - my_x, my_y, my_z)
    yp = (my_x, 1 - my_y, my_z)
    my_row0 = my_y * HALF
    fwd_row0 = (1 - my_y) * HALF
    mine_row0 = my_x * M + my_y * HALF

    barrier = pltpu.get_barrier_semaphore()
    for nbr in (xp, yp):
        pl.semaphore_signal(barrier, inc=1, device_id=nbr,
                            device_id_type=pl.DeviceIdType.MESH)
    pl.semaphore_wait(barrier, 2)

    x_rdmas = []
    for c in range(NC):
        r = pltpu.make_async_remote_copy(
            src_ref=theirs_ref.at[pl.ds(c * CH, CH), :],
            dst_ref=recv_x.at[c],
            send_sem=send_x_sems.at[c],
            recv_sem=recv_x_sems.at[c],
            device_id=xp,
            device_id_type=pl.DeviceIdType.MESH,
        )
        r.start()
        x_rdmas.append(r)

    def mine_cp(c):
        return pltpu.make_async_copy(
            partial_ref.at[pl.ds(mine_row0 + c * CH, CH), :],
            mine_buf.at[c % 2],
            mine_sems.at[c],
        )

    for c in range(min(2, NC)):
        mine_cp(c).start()

    out_copies = []
    y_rdmas = []
    for c in range(NC):
        s = c % NSTAGE
        if c >= NSTAGE:
            out_copies[c - NSTAGE].wait()
            y_rdmas[c - NSTAGE].wait_send()
        x_rdmas[c].wait_recv()
        mine_cp(c).wait()
        ysum = mine_buf[c % 2] + recv_x[c].astype(jnp.float32)
        ms = jnp.mean(ysum * ysum, axis=-1, keepdims=True)
        res = ysum * lax.rsqrt(ms + 1e-6) * gamma_ref[:, :]
        out_stage[s] = res.astype(jnp.bfloat16)
        if c + 2 < NC:
            mine_cp(c + 2).start()

        cp = pltpu.make_async_copy(
            out_stage.at[s],
            out_ref.at[pl.ds(my_row0 + c * CH, CH), :],
            out_copy_sems.at[c],
        )
        cp.start()
        out_copies.append(cp)

        r = pltpu.make_async_remote_copy(
            src_ref=out_stage.at[s],
            dst_ref=out_ref.at[pl.ds(my_row0 + c * CH, CH), :],
            send_sem=send_y_sems.at[c],
            recv_sem=recv_y_sems.at[c],
            device_id=yp,
            device_id_type=pl.DeviceIdType.MESH,
        )
        r.start()
        y_rdmas.append(r)

    for c in range(NC):
        x_rdmas[c].wait_send()
    for c in range(max(0, NC - NSTAGE), NC):
        out_copies[c].wait()
        y_rdmas[c].wait_send()
    for c in range(NC):
        recv = pltpu.make_async_remote_copy(
            src_ref=out_stage.at[c % NSTAGE],
            dst_ref=out_ref.at[pl.ds(fwd_row0 + c * CH, CH), :],
            send_sem=send_y_sems.at[c],
            recv_sem=recv_y_sems.at[c],
            device_id=yp,
            device_id_type=pl.DeviceIdType.MESH,
        )
        recv.wait_recv()


def kernel(partial, gamma):
    my_x = lax.axis_index("x")
    my_y = lax.axis_index("y")
    rows = partial.reshape(8192, D)

    theirs = lax.dynamic_slice(
        rows, ((1 - my_x) * M + my_y * HALF, 0), (HALF, D)
    ).astype(jnp.bfloat16)
    gamma2 = gamma.reshape(1, D)

    return pl.pallas_call(
        _body,
        in_specs=[
            pl.BlockSpec(memory_space=pl.ANY),
            pl.BlockSpec(memory_space=pl.ANY),
            pl.BlockSpec(memory_space=pltpu.VMEM),
        ],
        out_specs=pl.BlockSpec(memory_space=pl.ANY),
        out_shape=jax.ShapeDtypeStruct((M, D), jnp.bfloat16),
        scratch_shapes=[
            pltpu.VMEM((2, CH, D), jnp.float32),
            pltpu.VMEM((NC, CH, D), jnp.bfloat16),
            pltpu.VMEM((NSTAGE, CH, D), jnp.bfloat16),
            pltpu.SemaphoreType.DMA((NC,)),
            pltpu.SemaphoreType.DMA((NC,)),
            pltpu.SemaphoreType.DMA((NC,)),
            pltpu.SemaphoreType.DMA((NC,)),
            pltpu.SemaphoreType.DMA((NC,)),
            pltpu.SemaphoreType.DMA((NC,)),
        ],
        compiler_params=pltpu.CompilerParams(
            collective_id=0,
            vmem_limit_bytes=56 * 1024 * 1024,
        ),
    )(rows, theirs, gamma2)


# device time: 241175 ns/iter; 2.1862x vs baseline; 1.0474x over previous
import jax
import jax.numpy as jnp
from jax import lax
from jax.experimental import pallas as pl
from jax.experimental.pallas import tpu as pltpu

M = 4096
D = 4096
HALF = M // 2
CH = 128
NC = HALF // CH
NSTAGE = 4


def _body(partial_ref, theirs_ref, gamma_ref, out_ref,
          mine_buf, recv_x, out_stage,
          mine_sems, send_x_sems, recv_x_sems,
          out_copy_sems, send_y_sems, recv_y_sems):
    my_x = lax.axis_index("x")
    my_y = lax.axis_index("y")
    my_z = lax.axis_index("z")
    xp = (1 - my_x, my_y, my_z)
    yp = (my_x, 1 - my_y, my_z)
    my_row0 = my_y * HALF
    fwd_row0 = (1 - my_y) * HALF
    mine_row0 = my_x * M + my_y * HALF

    barrier = pltpu.get_barrier_semaphore()
    for nbr in (xp, yp):
        pl.semaphore_signal(barrier, inc=1, device_id=nbr,
                            device_id_type=pl.DeviceIdType.MESH)
    pl.semaphore_wait(barrier, 2)

    x_rdmas = []
    for c in range(NC):
        r = pltpu.make_async_remote_copy(
            src_ref=theirs_ref.at[pl.ds(c * CH, CH), :],
            dst_ref=recv_x.at[c],
            send_sem=send_x_sems.at[c],
            recv_sem=recv_x_sems.at[c],
            device_id=xp,
            device_id_type=pl.DeviceIdType.MESH,
        )
        r.start()
        x_rdmas.append(r)

    def mine_cp(c):
        return pltpu.make_async_copy(
            partial_ref.at[pl.ds(mine_row0 + c * CH, CH), :],
            mine_buf.at[c % 2],
            mine_sems.at[c],
        )

    for c in range(min(2, NC)):
        mine_cp(c).start()

    out_copies = []
    y_rdmas = []
    for c in range(NC):
        s = c % NSTAGE
        if c >= NSTAGE:
            out_copies[c - NSTAGE].wait()
            y_rdmas[c - NSTAGE].wait_send()
        x_rdmas[c].wait_recv()
        mine_cp(c).wait()
        ysum = mine_buf[c % 2] + recv_x[c].astype(jnp.float32)
        ms = jnp.mean(ysum * ysum, axis=-1, keepdims=True)
        res = ysum * lax.rsqrt(ms + 1e-6) * gamma_ref[:, :]
        out_stage[s] = res.astype(jnp.bfloat16)
        if c + 2 < NC:
            mine_cp(c + 2).start()

        cp = pltpu.make_async_copy(
            out_stage.at[s],
            out_ref.at[pl.ds(my_row0 + c * CH, CH), :],
            out_copy_sems.at[c],
        )
        cp.start()
        out_copies.append(cp)

        r = pltpu.make_async_remote_copy(
            src_ref=out_stage.at[s],
            dst_ref=out_ref.at[pl.ds(my_row0 + c * CH, CH), :],
            send_sem=send_y_sems.at[c],
            recv_sem=recv_y_sems.at[c],
            device_id=yp,
            device_id_type=pl.DeviceIdType.MESH,
        )
        r.start()
        y_rdmas.append(r)

    for c in range(NC):
        x_rdmas[c].wait_send()
    for c in range(max(0, NC - NSTAGE), NC):
        out_copies[c].wait()
        y_rdmas[c].wait_send()
    for c in range(NC):
        recv = pltpu.make_async_remote_copy(
            src_ref=out_stage.at[c % NSTAGE],
            dst_ref=out_ref.at[pl.ds(fwd_row0 + c * CH, CH), :],
            send_sem=send_y_sems.at[c],
            recv_sem=recv_y_sems.at[c],
            device_id=yp,
            device_id_type=pl.DeviceIdType.MESH,
        )
        recv.wait_recv()


def kernel(partial, gamma):
    my_x = lax.axis_index("x")
    my_y = lax.axis_index("y")
    rows = partial.reshape(8192, D)

    theirs = lax.dynamic_slice(
        rows, ((1 - my_x) * M + my_y * HALF, 0), (HALF, D)
    ).astype(jnp.bfloat16)
    gamma2 = gamma.reshape(1, D)

    return pl.pallas_call(
        _body,
        in_specs=[
            pl.BlockSpec(memory_space=pl.ANY),
            pl.BlockSpec(memory_space=pl.ANY),
            pl.BlockSpec(memory_space=pltpu.VMEM),
        ],
        out_specs=pl.BlockSpec(memory_space=pl.ANY),
        out_shape=jax.ShapeDtypeStruct((M, D), jnp.bfloat16),
        scratch_shapes=[
            pltpu.VMEM((2, CH, D), jnp.float32),
            pltpu.VMEM((NC, CH, D), jnp.bfloat16),
            pltpu.VMEM((NSTAGE, CH, D), jnp.bfloat16),
            pltpu.SemaphoreType.DMA((NC,)),
            pltpu.SemaphoreType.DMA((NC,)),
            pltpu.SemaphoreType.DMA((NC,)),
            pltpu.SemaphoreType.DMA((NC,)),
            pltpu.SemaphoreType.DMA((NC,)),
            pltpu.SemaphoreType.DMA((NC,)),
        ],
        compiler_params=pltpu.CompilerParams(
            collective_id=0,
            vmem_limit_bytes=56 * 1024 * 1024,
        ),
    )(rows, theirs, gamma2)


# device time: 223106 ns/iter; 2.3633x vs baseline; 1.0810x over previous
import jax
import jax.numpy as jnp
from jax import lax
from jax.experimental import pallas as pl
from jax.experimental.pallas import tpu as pltpu

M = 4096
D = 4096
HALF = M // 2
CH = 128
NC = HALF // CH
NSTAGE = 4
LOOKAHEAD = 3


def _body(partial_ref, gamma_ref, out_ref,
          mine_buf, th_buf, send_x, recv_x, out_stage,
          mine_sems, th_sems, send_x_sems, recv_x_sems,
          out_copy_sems, send_y_sems, recv_y_sems):
    my_x = lax.axis_index("x")
    my_y = lax.axis_index("y")
    my_z = lax.axis_index("z")
    xp = (1 - my_x, my_y, my_z)
    yp = (my_x, 1 - my_y, my_z)
    my_row0 = my_y * HALF
    fwd_row0 = (1 - my_y) * HALF
    mine_row0 = my_x * M + my_y * HALF
    theirs_row0 = (1 - my_x) * M + my_y * HALF

    barrier = pltpu.get_barrier_semaphore()
    for nbr in (xp, yp):
        pl.semaphore_signal(barrier, inc=1, device_id=nbr,
                            device_id_type=pl.DeviceIdType.MESH)
    pl.semaphore_wait(barrier, 2)

    def mine_cp(c):
        return pltpu.make_async_copy(
            partial_ref.at[pl.ds(mine_row0 + c * CH, CH), :],
            mine_buf.at[c % 2],
            mine_sems.at[c],
        )

    def th_cp(c):
        return pltpu.make_async_copy(
            partial_ref.at[pl.ds(theirs_row0 + c * CH, CH), :],
            th_buf.at[c % 2],
            th_sems.at[c],
        )

    x_rdmas = {}

    def send_side(j):
        th_cp(j).wait()
        send_x[j] = th_buf[j % 2].astype(jnp.bfloat16)
        if j + 2 < NC:
            th_cp(j + 2).start()
        r = pltpu.make_async_remote_copy(
            src_ref=send_x.at[j],
            dst_ref=recv_x.at[j],
            send_sem=send_x_sems.at[j],
            recv_sem=recv_x_sems.at[j],
            device_id=xp,
            device_id_type=pl.DeviceIdType.MESH,
        )
        r.start()
        x_rdmas[j] = r

    for c in range(min(2, NC)):
        th_cp(c).start()
        mine_cp(c).start()
    for j in range(min(LOOKAHEAD, NC)):
        send_side(j)

    out_copies = []
    y_rdmas = []
    for c in range(NC):
        if c + LOOKAHEAD < NC:
            send_side(c + LOOKAHEAD)
        s = c % NSTAGE
        if c >= NSTAGE:
            out_copies[c - NSTAGE].wait()
            y_rdmas[c - NSTAGE].wait_send()
        x_rdmas[c].wait_recv()
        mine_cp(c).wait()
        ysum = mine_buf[c % 2] + recv_x[c].astype(jnp.float32)
        ms = jnp.mean(ysum * ysum, axis=-1, keepdims=True)
        res = ysum * lax.rsqrt(ms + 1e-6) * gamma_ref[:, :]
        out_stage[s] = res.astype(jnp.bfloat16)
        if c + 2 < NC:
            mine_cp(c + 2).start()

        cp = pltpu.make_async_copy(
            out_stage.at[s],
            out_ref.at[pl.ds(my_row0 + c * CH, CH), :],
            out_copy_sems.at[c],
        )
        cp.start()
        out_copies.append(cp)

        r = pltpu.make_async_remote_copy(
            src_ref=out_stage.at[s],
            dst_ref=out_ref.at[pl.ds(my_row0 + c * CH, CH), :],
            send_sem=send_y_sems.at[c],
            recv_sem=recv_y_sems.at[c],
            device_id=yp,
            device_id_type=pl.DeviceIdType.MESH,
        )
        r.start()
        y_rdmas.append(r)

    for c in range(NC):
        x_rdmas[c].wait_send()
    for c in range(max(0, NC - NSTAGE), NC):
        out_copies[c].wait()
        y_rdmas[c].wait_send()
    for c in range(NC):
        recv = pltpu.make_async_remote_copy(
            src_ref=out_stage.at[c % NSTAGE],
            dst_ref=out_ref.at[pl.ds(fwd_row0 + c * CH, CH), :],
            send_sem=send_y_sems.at[c],
            recv_sem=recv_y_sems.at[c],
            device_id=yp,
            device_id_type=pl.DeviceIdType.MESH,
        )
        recv.wait_recv()


def kernel(partial, gamma):
    rows = partial.reshape(8192, D)
    gamma2 = gamma.reshape(1, D)

    return pl.pallas_call(
        _body,
        in_specs=[
            pl.BlockSpec(memory_space=pl.ANY),
            pl.BlockSpec(memory_space=pltpu.VMEM),
        ],
        out_specs=pl.BlockSpec(memory_space=pl.ANY),
        out_shape=jax.ShapeDtypeStruct((M, D), jnp.bfloat16),
        scratch_shapes=[
            pltpu.VMEM((2, CH, D), jnp.float32),
            pltpu.VMEM((2, CH, D), jnp.float32),
            pltpu.VMEM((NC, CH, D), jnp.bfloat16),
            pltpu.VMEM((NC, CH, D), jnp.bfloat16),
            pltpu.VMEM((NSTAGE, CH, D), jnp.bfloat16),
            pltpu.SemaphoreType.DMA((NC,)),
            pltpu.SemaphoreType.DMA((NC,)),
            pltpu.SemaphoreType.DMA((NC,)),
            pltpu.SemaphoreType.DMA((NC,)),
            pltpu.SemaphoreType.DMA((NC,)),
            pltpu.SemaphoreType.DMA((NC,)),
            pltpu.SemaphoreType.DMA((NC,)),
        ],
        compiler_params=pltpu.CompilerParams(
            collective_id=0,
            vmem_limit_bytes=56 * 1024 * 1024,
        ),
    )(rows, gamma2)
